# Initial kernel scaffold; baseline (speedup 1.0000x reference)
import jax
import jax.numpy as jnp
from jax.experimental import pallas as pl


def kernel(x0, x1, selected_indices):
    raise NotImplementedError("write your pallas kernel here")


# speedup vs baseline: 4.2228x; 4.2228x over previous
import functools

import jax
import jax.numpy as jnp
from jax import lax
from jax.experimental import pallas as pl
from jax.experimental.pallas import tpu as pltpu

_B, _N, _NC, _NM, _PH, _PW = 16, 20000, 80, 32, 160, 160
_ND = 100
_ROW = 5 + _NC + _NM
_PHW = _PH * _PW
_CW = 512
_KB = _B * _NM


def _prep_body(idx_ref, xb_ref, x_ref, hdr_ref, scat_ref):
    del idx_ref
    i = pl.program_id(0)
    row = x_ref[0]
    conf = row[:, 4:5]
    sc = row[:, 5:5 + _NC] * conf
    msc = jnp.max(sc, axis=1, keepdims=True)
    io = lax.broadcasted_iota(jnp.int32, (1, _NC), 1)
    cat = jnp.min(jnp.where(sc == msc, io, _NC), axis=1, keepdims=True)
    bx = row[:, 0:1]
    by = row[:, 1:2]
    bw = row[:, 2:3]
    bh = row[:, 3:4]
    x1 = bx - 0.5 * bw
    y1 = by - 0.5 * bh
    x2 = bx + 0.5 * bw
    y2 = by + 0.5 * bh
    xb = xb_ref[i]
    xf = jnp.full((1, 1), 0.0, jnp.float32) + xb.astype(jnp.float32)
    zero = jnp.zeros((1, 1), jnp.float32)
    hdr = jnp.concatenate(
        [xf, x1, y1, x2, y2, cat.astype(jnp.float32), msc, zero], axis=1)
    hdr_ref[0] = hdr
    mask_sel = row[:, 5 + _NC:]
    b16 = lax.broadcasted_iota(jnp.int32, (_B, _NM), 0)
    scat_ref[0] = jnp.where(b16 == xb, mask_sel, 0.0)


def _main_body(s_ref, hdr_ref, p_ref, o_ref):
    j = pl.program_id(0)
    s = s_ref[...]
    p = p_ref[...]
    m = jnp.dot(s, p, preferred_element_type=jnp.float32)
    m = 1.0 / (1.0 + jnp.exp(-m))
    g = j * _CW + lax.broadcasted_iota(jnp.int32, (_ND, _CW), 1)
    h = g // _PW
    w = g - h * _PW
    rf = w.astype(jnp.float32)
    cf = h.astype(jnp.float32)
    db = hdr_ref[...] * 0.25
    x1b = db[:, 1:2]
    y1b = db[:, 2:3]
    x2b = db[:, 3:4]
    y2b = db[:, 4:5]
    crop = ((rf >= x1b) & (rf < x2b) & (cf >= y1b) & (cf < y2b))
    o_ref[...] = m * crop.astype(jnp.float32)


def _run(x0, x1, selected_indices, interpret=False):
    xsel = selected_indices[:, 0]
    ysel = selected_indices[:, 2]
    flat_idx = xsel * _N + ysel
    x0r = x0.reshape(_B * _N, 1, _ROW)

    hdr, scat = pl.pallas_call(
        _prep_body,
        grid_spec=pltpu.PrefetchScalarGridSpec(
            num_scalar_prefetch=2,
            grid=(_ND,),
            in_specs=[
                pl.BlockSpec((1, 1, _ROW), lambda i, idx, xb: (idx[i], 0, 0)),
            ],
            out_specs=[
                pl.BlockSpec((1, 1, 8), lambda i, idx, xb: (i, 0, 0)),
                pl.BlockSpec((1, _B, _NM), lambda i, idx, xb: (i, 0, 0)),
            ],
        ),
        out_shape=[
            jax.ShapeDtypeStruct((_ND, 1, 8), jnp.float32),
            jax.ShapeDtypeStruct((_ND, _B, _NM), jnp.float32),
        ],
        interpret=interpret,
    )(flat_idx, xsel, x0r)

    hdr2 = hdr.reshape(_ND, 8)
    s = scat.reshape(_ND, _KB)
    p = x1.reshape(_KB, _PHW)

    masks = pl.pallas_call(
        _main_body,
        grid=(_PHW // _CW,),
        in_specs=[
            pl.BlockSpec((_ND, _KB), lambda j: (0, 0)),
            pl.BlockSpec((_ND, 8), lambda j: (0, 0)),
            pl.BlockSpec((_KB, _CW), lambda j: (0, j)),
        ],
        out_specs=pl.BlockSpec((_ND, _CW), lambda j: (0, j)),
        out_shape=jax.ShapeDtypeStruct((_ND, _PHW), jnp.float32),
        interpret=interpret,
    )(s, hdr2, p)

    return jnp.concatenate([hdr2[:, :7], masks], axis=1)


@jax.jit
def kernel(x0, x1, selected_indices):
    return _run(x0, x1, selected_indices)


# speedup vs baseline: 9.8623x; 2.3355x over previous
import functools

import jax
import jax.numpy as jnp
from jax import lax
from jax.experimental import pallas as pl
from jax.experimental.pallas import tpu as pltpu

_B, _N, _NC, _NM, _PH, _PW = 16, 20000, 80, 32, 160, 160
_ND = 100
_ROW = 5 + _NC + _NM
_PHW = _PH * _PW
_KB = _B * _NM
_HB = 8


def _prep_body(x_idx_ref, y_ref, x_ref, hdr_ref, scat_ref):
    i = pl.program_id(0)
    rows8 = x_ref[0]
    ymod = y_ref[i] % 8
    sel = lax.broadcasted_iota(jnp.int32, (8, _ROW), 0) == ymod
    row = jnp.sum(jnp.where(sel, rows8, 0.0), axis=0, keepdims=True)
    conf = row[:, 4:5]
    sc = row[:, 5:5 + _NC] * conf
    msc = jnp.max(sc, axis=1, keepdims=True)
    io = lax.broadcasted_iota(jnp.int32, (1, _NC), 1)
    cat = jnp.min(jnp.where(sc == msc, io, _NC), axis=1, keepdims=True)
    bx = row[:, 0:1]
    by = row[:, 1:2]
    bw = row[:, 2:3]
    bh = row[:, 3:4]
    x1 = bx - 0.5 * bw
    y1 = by - 0.5 * bh
    x2 = bx + 0.5 * bw
    y2 = by + 0.5 * bh
    xb = x_idx_ref[i]
    xf = jnp.zeros((1, 1), jnp.float32) + xb.astype(jnp.float32)
    zero = jnp.zeros((1, 1), jnp.float32)
    hdr = jnp.concatenate(
        [xf, x1, y1, x2, y2, cat.astype(jnp.float32), msc, zero], axis=1)
    hdr_ref[0] = hdr
    mask_sel = row[:, 5 + _NC:]
    b16 = lax.broadcasted_iota(jnp.int32, (_B, _NM), 0)
    scat_ref[0] = jnp.where(b16 == xb, mask_sel, 0.0)


def _main_body(s_ref, hdr_ref, p_ref, o_ref):
    jh = pl.program_id(0)
    s = s_ref[...]
    p4 = p_ref[...]
    p3 = p4.reshape(_KB, _HB, _PW)
    db = hdr_ref[...] * 0.25
    x1b = db[:, 1:2]
    y1b = db[:, 2:3]
    x2b = db[:, 3:4]
    y2b = db[:, 4:5]
    rf = lax.broadcasted_iota(jnp.int32, (_ND, _PW), 1).astype(jnp.float32)
    colmask = (rf >= x1b) & (rf < x2b)
    for t in range(_HB):
        pt = p3[:, t, :]
        m = jnp.dot(s, pt, preferred_element_type=jnp.float32)
        m = 1.0 / (1.0 + jnp.exp(-m))
        cf = (jh * _HB + t).astype(jnp.float32)
        rowmask = (cf >= y1b) & (cf < y2b)
        o_ref[:, t, :] = m * (colmask & rowmask).astype(jnp.float32)


def _run(x0, x1, selected_indices, interpret=False):
    xsel = selected_indices[:, 0]
    ysel = selected_indices[:, 2]

    hdr, scat = pl.pallas_call(
        _prep_body,
        grid_spec=pltpu.PrefetchScalarGridSpec(
            num_scalar_prefetch=2,
            grid=(_ND,),
            in_specs=[
                pl.BlockSpec((1, 8, _ROW),
                             lambda i, xs, ys: (xs[i], ys[i] // 8, 0)),
            ],
            out_specs=[
                pl.BlockSpec((1, 1, 8), lambda i, xs, ys: (i, 0, 0)),
                pl.BlockSpec((1, _B, _NM), lambda i, xs, ys: (i, 0, 0)),
            ],
        ),
        out_shape=[
            jax.ShapeDtypeStruct((_ND, 1, 8), jnp.float32),
            jax.ShapeDtypeStruct((_ND, _B, _NM), jnp.float32),
        ],
        interpret=interpret,
    )(xsel, ysel, x0)

    hdr2 = hdr.reshape(_ND, 8)
    s = scat.reshape(_ND, _KB)

    masks = pl.pallas_call(
        _main_body,
        grid=(_PH // _HB,),
        in_specs=[
            pl.BlockSpec((_ND, _KB), lambda j: (0, 0)),
            pl.BlockSpec((_ND, 8), lambda j: (0, 0)),
            pl.BlockSpec((_B, _NM, _HB, _PW), lambda j: (0, 0, j, 0)),
        ],
        out_specs=pl.BlockSpec((_ND, _HB, _PW), lambda j: (0, j, 0)),
        out_shape=jax.ShapeDtypeStruct((_ND, _PH, _PW), jnp.float32),
        interpret=interpret,
    )(s, hdr2, x1)

    return jnp.concatenate([hdr2[:, :7], masks.reshape(_ND, _PHW)], axis=1)


@jax.jit
def kernel(x0, x1, selected_indices):
    return _run(x0, x1, selected_indices)
